# baseline (speedup 1.0000x reference)
import functools

import jax
import jax.numpy as jnp
from jax import lax
from jax.experimental import pallas as pl
from jax.experimental.pallas import tpu as pltpu
from jax.experimental.pallas import tpu_sc as plsc

_N = 32768
_D = 2048
_K = 1638
_RANK = _D - _K
_NC = 2
_NS = 16
_NW = _NC * _NS
_GROUP = 16
_PITCH = _D + 1


def _sc_body(x_hbm, o_hbm, xin, keys, outb, hist, sem):
    lane = lax.iota(jnp.int32, 16)
    ones = jnp.ones((16,), jnp.int32)
    zeros_i = jnp.zeros((16,), jnp.int32)
    rank0 = jnp.full((16,), _RANK, jnp.int32)

    wid = lax.axis_index("s") * _NC + lax.axis_index("c")
    rows_per_worker = _N // _NW
    groups = rows_per_worker // _GROUP
    base = wid * rows_per_worker

    def _z(j, c):
        hist[j, :] = zeros_i
        return c
    lax.fori_loop(0, 256, _z, 0)

    def scan_level(rvec):
        def sc(j, c):
            cum, nb, cumlt = c
            h = hist[j, :]
            hist[j, :] = zeros_i
            cum2 = cum + h
            le = cum2 <= rvec
            nb2 = nb + jnp.where(le, 1, 0)
            cumlt2 = jnp.where(le, cum2, cumlt)
            return cum2, nb2, cumlt2
        _, nb, cumlt = lax.fori_loop(0, 256, sc, (zeros_i, zeros_i, zeros_i))
        return nb, rvec - cumlt

    def group_body(g, c):
        row0 = base + g * _GROUP

        copies = [
            pltpu.async_copy(x_hbm.at[row0 + r], xin.at[r, pl.ds(0, _D)], sem)
            for r in range(_GROUP)
        ]
        for h in copies:
            h.wait()

        def p1(j, acc):
            jj = jnp.full((16,), j, jnp.int32)
            xv = plsc.load_gather(xin, [lane, jj])
            acc = acc + xv * xv
            b = lax.bitcast_convert_type(xv, jnp.int32)
            s = jnp.right_shift(b, 31)
            kb = jnp.bitwise_xor(b, jnp.bitwise_and(s, jnp.int32(0x7FFFFFFF)))
            keys[j, :] = kb
            d1 = jnp.right_shift(kb, 24) + 128
            plsc.addupdate_scatter(hist, [d1, lane], ones)
            return acc

        sumsq = lax.fori_loop(0, _D, p1, jnp.zeros((16,), jnp.float32))

        b1, r1 = scan_level(rank0)
        b1s = b1 - 128

        def p2(j, c):
            kb = keys[j, :]
            m = jnp.right_shift(kb, 24) == b1s
            d2 = jnp.bitwise_and(jnp.right_shift(kb, 16), 255)
            plsc.addupdate_scatter(hist, [d2, lane], ones, mask=m)
            return c
        lax.fori_loop(0, _D, p2, 0)

        b2, r2 = scan_level(r1)
        p2s = b1s * 256 + b2

        def p3(j, c):
            kb = keys[j, :]
            m = jnp.right_shift(kb, 16) == p2s
            d3 = jnp.bitwise_and(jnp.right_shift(kb, 8), 255)
            plsc.addupdate_scatter(hist, [d3, lane], ones, mask=m)
            return c
        lax.fori_loop(0, _D, p3, 0)

        b3, _ = scan_level(r2)
        p3s = p2s * 256 + b3
        ks_t = p3s * 256

        st = jnp.right_shift(ks_t, 31)
        tbits = jnp.bitwise_xor(ks_t, jnp.bitwise_and(st, jnp.int32(0x7FFFFFFF)))
        tf = lax.bitcast_convert_type(tbits, jnp.float32)

        sb = lax.bitcast_convert_type(sumsq, jnp.int32)
        y = lax.bitcast_convert_type(
            jnp.int32(0x5F3759DF) - jnp.right_shift(sb, 1), jnp.float32)
        half = 0.5 * sumsq
        for _ in range(3):
            y = y * (1.5 - half * y * y)
        boost = jnp.exp(jnp.float32(_K) * y)

        def po(j, c):
            jj = jnp.full((16,), j, jnp.int32)
            xv = plsc.load_gather(xin, [lane, jj])
            ov = jnp.where(xv >= tf, xv * boost, jnp.float32(0.0))
            plsc.store_scatter(outb, [lane, jj], ov)
            return c
        lax.fori_loop(0, _D, po, 0)

        copies = [
            pltpu.async_copy(outb.at[r, pl.ds(0, _D)], o_hbm.at[row0 + r], sem)
            for r in range(_GROUP)
        ]
        for h in copies:
            h.wait()
        return c

    lax.fori_loop(0, groups, group_body, 0)


@functools.partial(jax.jit, static_argnames=())
def kernel(inputs):
    n, d = inputs.shape
    assert (n, d) == (_N, _D)
    mesh = plsc.VectorSubcoreMesh(
        core_axis_name="c", subcore_axis_name="s",
        num_cores=_NC, num_subcores=_NS)
    f = pl.kernel(
        _sc_body,
        out_type=jax.ShapeDtypeStruct((_N, _D), jnp.float32),
        mesh=mesh,
        scratch_types=[
            pltpu.VMEM((_GROUP, _PITCH), jnp.float32),
            pltpu.VMEM((_D, 16), jnp.int32),
            pltpu.VMEM((_GROUP, _PITCH), jnp.float32),
            pltpu.VMEM((256, 16), jnp.int32),
            pltpu.SemaphoreType.DMA,
        ],
        compiler_params=pltpu.CompilerParams(
            use_tc_tiling_on_sc=False, needs_layout_passes=False),
    )
    return f(inputs)


# speedup vs baseline: 3.8647x; 3.8647x over previous
import functools

import jax
import jax.numpy as jnp
from jax import lax
from jax.experimental import pallas as pl
from jax.experimental.pallas import tpu as pltpu
from jax.experimental.pallas import tpu_sc as plsc

_N = 32768
_D = 2048
_K = 1638
_RANK = _D - _K
_NC = 2
_NS = 16
_NW = _NC * _NS
_GROUP = 16
_PITCH = _D + 1
_UNROLL = 8


def _key(xv):
    b = lax.bitcast_convert_type(xv, jnp.int32)
    s = jnp.right_shift(b, 31)
    return jnp.bitwise_xor(b, jnp.bitwise_and(s, jnp.int32(0x7FFFFFFF)))


def _sc_body(x_hbm, o_hbm, xin, outb, hist, ssq, in_sem, out_sem):
    lane = lax.iota(jnp.int32, 16)
    ones = jnp.ones((16,), jnp.int32)
    zeros_i = jnp.zeros((16,), jnp.int32)
    rank0 = jnp.full((16,), _RANK, jnp.int32)

    wid = lax.axis_index("s") * _NC + lax.axis_index("c")
    rows_per_worker = _N // _NW
    groups = rows_per_worker // _GROUP
    base = wid * rows_per_worker

    def in_copies(g):
        par16 = (g & 1) * _GROUP
        row0 = base + g * _GROUP
        return [(x_hbm.at[row0 + r], xin.at[par16 + r, pl.ds(0, _D)])
                for r in range(_GROUP)]

    def out_copies(g):
        row0 = base + g * _GROUP
        return [(outb.at[r, pl.ds(0, _D)], o_hbm.at[row0 + r])
                for r in range(_GROUP)]

    def _z(j, c):
        hist[j, :] = zeros_i
        return c
    lax.fori_loop(0, 256, _z, 0)

    def scan_level(rvec):
        @plsc.parallel_loop(0, 256, 1, unroll=4,
                            carry=(zeros_i, zeros_i, zeros_i))
        def res(j, c):
            cum, nb, cumlt = c
            h = hist[j, :]
            hist[j, :] = zeros_i
            cum2 = cum + h
            le = cum2 <= rvec
            return cum2, nb + jnp.where(le, 1, 0), jnp.where(le, cum2, cumlt)
        _, nb, cumlt = res
        return nb, rvec - cumlt

    for src, dst in in_copies(0):
        pltpu.async_copy(src, dst, in_sem)

    def group_body(g, c):
        par16 = (g & 1) * _GROUP
        rowsel = lane + par16

        for src, dst in in_copies(g):
            pltpu.make_async_copy(src, dst, in_sem).wait()

        @pl.when(g + 1 < groups)
        def _prefetch():
            for src, dst in in_copies(g + 1):
                pltpu.async_copy(src, dst, in_sem)

        @plsc.parallel_loop(0, _D, 1, unroll=_UNROLL)
        def _p1(j):
            jj = jnp.full((16,), j, jnp.int32)
            kb = _key(plsc.load_gather(xin, [rowsel, jj]))
            d1 = jnp.right_shift(kb, 24) + 128
            plsc.addupdate_scatter(hist, [d1, lane], ones)

        b1, r1 = scan_level(rank0)
        b1s = b1 - 128

        ssq[...] = jnp.zeros((16,), jnp.float32)

        @plsc.parallel_loop(0, _D, 1, unroll=_UNROLL)
        def _p2(j):
            jj = jnp.full((16,), j, jnp.int32)
            xv = plsc.load_gather(xin, [rowsel, jj])
            plsc.addupdate(ssq.at[pl.ds(0, 16)], xv * xv)
            kb = _key(xv)
            m = jnp.right_shift(kb, 24) == b1s
            d2 = jnp.bitwise_and(jnp.right_shift(kb, 16), 255)
            plsc.addupdate_scatter(hist, [d2, lane], ones, mask=m)

        b2, r2 = scan_level(r1)
        p2s = b1s * 256 + b2

        @plsc.parallel_loop(0, _D, 1, unroll=_UNROLL)
        def _p3(j):
            jj = jnp.full((16,), j, jnp.int32)
            kb = _key(plsc.load_gather(xin, [rowsel, jj]))
            m = jnp.right_shift(kb, 16) == p2s
            d3 = jnp.bitwise_and(jnp.right_shift(kb, 8), 255)
            plsc.addupdate_scatter(hist, [d3, lane], ones, mask=m)

        b3, _ = scan_level(r2)
        p3s = p2s * 256 + b3
        ks_t = p3s * 256

        tf = lax.bitcast_convert_type(
            jnp.bitwise_xor(
                ks_t,
                jnp.bitwise_and(jnp.right_shift(ks_t, 31),
                                jnp.int32(0x7FFFFFFF))),
            jnp.float32)

        s = ssq[...]
        y = lax.bitcast_convert_type(
            jnp.int32(0x5F3759DF)
            - jnp.right_shift(lax.bitcast_convert_type(s, jnp.int32), 1),
            jnp.float32)
        half = 0.5 * s
        for _ in range(3):
            y = y * (1.5 - half * y * y)
        boost = jnp.exp(jnp.float32(_K) * y)

        @pl.when(g > 0)
        def _drain_out():
            for src, dst in out_copies(g - 1):
                pltpu.make_async_copy(src, dst, out_sem).wait()

        @plsc.parallel_loop(0, _D, 1, unroll=_UNROLL)
        def _po(j):
            jj = jnp.full((16,), j, jnp.int32)
            xv = plsc.load_gather(xin, [rowsel, jj])
            ov = jnp.where(xv >= tf, xv * boost, jnp.float32(0.0))
            plsc.store_scatter(outb, [lane, jj], ov)

        for src, dst in out_copies(g):
            pltpu.async_copy(src, dst, out_sem)
        return c

    lax.fori_loop(0, groups, group_body, 0)

    for src, dst in out_copies(groups - 1):
        pltpu.make_async_copy(src, dst, out_sem).wait()


@functools.partial(jax.jit, static_argnames=())
def kernel(inputs):
    n, d = inputs.shape
    assert (n, d) == (_N, _D)
    mesh = plsc.VectorSubcoreMesh(
        core_axis_name="c", subcore_axis_name="s",
        num_cores=_NC, num_subcores=_NS)
    f = pl.kernel(
        _sc_body,
        out_type=jax.ShapeDtypeStruct((_N, _D), jnp.float32),
        mesh=mesh,
        scratch_types=[
            pltpu.VMEM((2 * _GROUP, _PITCH), jnp.float32),
            pltpu.VMEM((_GROUP, _PITCH), jnp.float32),
            pltpu.VMEM((256, 16), jnp.int32),
            pltpu.VMEM((16,), jnp.float32),
            pltpu.SemaphoreType.DMA,
            pltpu.SemaphoreType.DMA,
        ],
        compiler_params=pltpu.CompilerParams(
            use_tc_tiling_on_sc=False, needs_layout_passes=False),
    )
    return f(inputs)
